# baseline (speedup 1.0000x reference)
import functools

import jax
import jax.numpy as jnp
from jax import lax
from jax.experimental import pallas as pl
from jax.experimental.pallas import tpu as pltpu
from jax.experimental.pallas import tpu_sc as plsc

_DIM = 64


_info = plsc.get_sparse_core_info()
_NC, _NS = _info.num_cores, _info.num_subcores
_NW = _NC * _NS

_SUB = 128
_CHUNK = 512


def _gather_body(idx_hbm, table_hbm, out_hbm, idx_v, rows_v, gsem, bpw):
    wid = lax.axis_index("s") * _NC + lax.axis_index("c")
    base = wid * bpw
    pltpu.sync_copy(idx_hbm.at[pl.ds(base, bpw)], idx_v)

    def chunk(ci, _):
        off = ci * _CHUNK
        handles = []
        for j in range(_CHUNK // _SUB):
            handles.append(pltpu.async_copy(
                table_hbm.at[idx_v.at[pl.ds(off + j * _SUB, _SUB)]],
                rows_v.at[pl.ds(j * _SUB, _SUB)],
                gsem,
            ))
        for h in handles:
            h.wait()
        pltpu.sync_copy(rows_v, out_hbm.at[pl.ds(base + off, _CHUNK)])
        return ()

    lax.fori_loop(0, bpw // _CHUNK, chunk, (), unroll=False)


def _sc_gather(idx_flat, table):
    n = idx_flat.shape[0]
    assert n % (_NW * _CHUNK) == 0
    bpw = n // _NW
    mesh = plsc.VectorSubcoreMesh(core_axis_name="c", subcore_axis_name="s")
    body = functools.partial(_gather_body, bpw=bpw)
    return pl.kernel(
        body,
        out_type=jax.ShapeDtypeStruct((n, _DIM), jnp.float32),
        mesh=mesh,
        scratch_types=[
            pltpu.VMEM((bpw,), jnp.int32),
            pltpu.VMEM((_CHUNK, _DIM), jnp.float32),
            pltpu.SemaphoreType.DMA,
        ],
        compiler_params=pltpu.CompilerParams(use_tc_tiling_on_sc=False),
    )(idx_flat, table)


_ROWS = 4096


def _proj_body(emb_ref, w_ref, b_ref, out_ref):
    acc = jnp.dot(emb_ref[...], w_ref[...], preferred_element_type=jnp.float32)
    acc = acc + b_ref[...]
    out_ref[...] = acc * jax.nn.sigmoid(acc)


def _tc_project(emb, W, b2d):
    n = emb.shape[0]
    return pl.pallas_call(
        _proj_body,
        grid=(n // _ROWS,),
        in_specs=[
            pl.BlockSpec((_ROWS, _DIM), lambda i: (i, 0)),
            pl.BlockSpec((_DIM, _DIM), lambda i: (0, 0)),
            pl.BlockSpec((1, _DIM), lambda i: (0, 0)),
        ],
        out_specs=pl.BlockSpec((_ROWS, _DIM), lambda i: (i, 0)),
        out_shape=jax.ShapeDtypeStruct((n, _DIM), jnp.float32),
    )(emb, W, b2d)


def kernel(x, table, W, b):
    B, F = x.shape
    idx_flat = x.reshape(-1)
    emb = _sc_gather(idx_flat, table)
    out = _tc_project(emb, W, b.reshape(1, _DIM))
    return out.reshape(B, F, _DIM)


# speedup vs baseline: 1.2742x; 1.2742x over previous
import functools

import jax
import jax.numpy as jnp
from jax import lax
from jax.experimental import pallas as pl
from jax.experimental.pallas import tpu as pltpu
from jax.experimental.pallas import tpu_sc as plsc

_DIM = 64


_info = plsc.get_sparse_core_info()
_NC, _NS = _info.num_cores, _info.num_subcores
_NW = _NC * _NS

_SUB = 128
_CHUNK = 512


def _gather_body(idx_hbm, table_hbm, out_hbm, idx_v, rows_v, gsem, bpw):
    wid = lax.axis_index("s") * _NC + lax.axis_index("c")
    base = wid * bpw
    pltpu.sync_copy(idx_hbm.at[pl.ds(base, bpw)], idx_v)

    def chunk(ci, _):
        off = ci * _CHUNK
        handles = []
        for j in range(_CHUNK // _SUB):
            handles.append(pltpu.async_copy(
                table_hbm.at[idx_v.at[pl.ds(off + j * _SUB, _SUB)]],
                rows_v.at[pl.ds(j * _SUB, _SUB)],
                gsem,
            ))
        for h in handles:
            h.wait()
        pltpu.sync_copy(rows_v, out_hbm.at[pl.ds(base + off, _CHUNK)])
        return ()

    lax.fori_loop(0, bpw // _CHUNK, chunk, (), unroll=False)


def _sc_gather(idx_flat, table):
    n = idx_flat.shape[0]
    assert n % (_NW * _CHUNK) == 0
    bpw = n // _NW
    mesh = plsc.VectorSubcoreMesh(core_axis_name="c", subcore_axis_name="s")
    body = functools.partial(_gather_body, bpw=bpw)
    return pl.kernel(
        body,
        out_type=jax.ShapeDtypeStruct((n, _DIM), jnp.float32),
        mesh=mesh,
        scratch_types=[
            pltpu.VMEM((bpw,), jnp.int32),
            pltpu.VMEM((_CHUNK, _DIM), jnp.float32),
            pltpu.SemaphoreType.DMA,
        ],
        compiler_params=pltpu.CompilerParams(use_tc_tiling_on_sc=False),
    )(idx_flat, table)


_ROWS = 2048


def _proj_body(emb_ref, w_ref, b_ref, out_ref):
    e = emb_ref[0]
    acc = lax.dot_general(
        w_ref[...], e, (((0,), (1,)), ((), ())),
        preferred_element_type=jnp.float32,
    )
    acc = acc + b_ref[...]
    out_ref[0] = acc * jax.nn.sigmoid(acc)


def _tc_project(emb3, W, bcol):
    F, B = emb3.shape[0], emb3.shape[1]
    return pl.pallas_call(
        _proj_body,
        grid=(F, B // _ROWS),
        in_specs=[
            pl.BlockSpec((1, _ROWS, _DIM), lambda f, i: (f, i, 0)),
            pl.BlockSpec((_DIM, _DIM), lambda f, i: (0, 0)),
            pl.BlockSpec((_DIM, 1), lambda f, i: (0, 0)),
        ],
        out_specs=pl.BlockSpec((1, _DIM, _ROWS), lambda f, i: (f, 0, i)),
        out_shape=jax.ShapeDtypeStruct((F, _DIM, B), jnp.float32),
    )(emb3, W, bcol)


def kernel(x, table, W, b):
    B, F = x.shape
    idx_flat = x.T.reshape(-1)
    emb = _sc_gather(idx_flat, table)
    emb3 = emb.reshape(F, B, _DIM)
    out3 = _tc_project(emb3, W, b.reshape(_DIM, 1))
    return out3.transpose(2, 0, 1)
